# baseline (speedup 1.0000x reference)
import functools

import jax
import jax.numpy as jnp
from jax import lax
from jax.experimental import pallas as pl
from jax.experimental.pallas import tpu as pltpu
from jax.experimental.pallas import tpu_sc as plsc

_INFO = plsc.get_sparse_core_info()
_NC, _NS = _INFO.num_cores, _INFO.num_subcores
_NW = _NC * _NS
_CHUNK_ROWS = 16
_NBUF = 3
_SC_ROWS = 2048
_BLOCK_ROWS = 1024


def _make_sc_copy(rows, dim, dtype):
    rows_per_w = rows // _NW
    n_chunks = rows_per_w // _CHUNK_ROWS
    mesh = plsc.VectorSubcoreMesh(core_axis_name="c", subcore_axis_name="s")

    @functools.partial(
        pl.kernel,
        mesh=mesh,
        out_type=jax.ShapeDtypeStruct((rows, dim), dtype),
        scratch_types=(
            [pltpu.VMEM((_CHUNK_ROWS, dim), dtype)] * _NBUF
            + [pltpu.SemaphoreType.DMA] * (2 * _NBUF)
        ),
    )
    def sc_copy(tab, out, *refs):
        bufs = refs[:_NBUF]
        lsems = refs[_NBUF : 2 * _NBUF]
        ssems = refs[2 * _NBUF :]
        wid = lax.axis_index("s") * _NC + lax.axis_index("c")
        base = wid * rows_per_w

        def src(i):
            return tab.at[pl.ds(base + i * _CHUNK_ROWS, _CHUNK_ROWS)]

        def dst(i):
            return out.at[pl.ds(base + i * _CHUNK_ROWS, _CHUNK_ROWS)]

        loads = [None] * n_chunks
        stores = [None] * n_chunks
        for i in range(min(_NBUF, n_chunks)):
            loads[i] = pltpu.async_copy(src(i), bufs[i], lsems[i])
        for i in range(n_chunks):
            b = i % _NBUF
            loads[i].wait()
            stores[i] = pltpu.async_copy(bufs[b], dst(i), ssems[b])
            nxt = i + _NBUF
            if nxt < n_chunks:
                stores[i].wait()
                loads[nxt] = pltpu.async_copy(src(nxt), bufs[b], lsems[b])
        for i in range(max(0, n_chunks - _NBUF), n_chunks):
            if stores[i] is not None and i + _NBUF >= n_chunks:
                stores[i].wait()

    return sc_copy


def _tc_body(src, dst):
    dst[...] = src[...]


def _tc_copy(rows, dim, dtype, tab):
    return pl.pallas_call(
        _tc_body,
        out_shape=jax.ShapeDtypeStruct((rows, dim), dtype),
        grid=(rows // _BLOCK_ROWS,),
        in_specs=[pl.BlockSpec((_BLOCK_ROWS, dim), lambda i: (i, 0))],
        out_specs=pl.BlockSpec((_BLOCK_ROWS, dim), lambda i: (i, 0)),
    )(tab)


def kernel(x, emb_weight):
    sl = x.shape[1]
    dim = emb_weight.shape[1]
    dtype = emb_weight.dtype
    tc_rows = sl - _SC_ROWS
    sc_part = _make_sc_copy(_SC_ROWS, dim, dtype)(emb_weight[tc_rows:sl])
    tc_part = _tc_copy(tc_rows, dim, dtype, emb_weight[:tc_rows])
    return jnp.concatenate([tc_part, sc_part], axis=0)


# speedup vs baseline: 1.1792x; 1.1792x over previous
import functools

import jax
import jax.numpy as jnp
from jax import lax
from jax.experimental import pallas as pl
from jax.experimental.pallas import tpu as pltpu
from jax.experimental.pallas import tpu_sc as plsc

_INFO = plsc.get_sparse_core_info()
_NC, _NS = _INFO.num_cores, _INFO.num_subcores
_NW = _NC * _NS
_CHUNK_ROWS = 16
_NBUF = 3
_SC_ROWS = 3072
_BLOCK_ROWS = 1024


def _make_sc_copy(rows, dim, dtype):
    rows_per_w = rows // _NW
    n_chunks = rows_per_w // _CHUNK_ROWS
    mesh = plsc.VectorSubcoreMesh(core_axis_name="c", subcore_axis_name="s")

    @functools.partial(
        pl.kernel,
        mesh=mesh,
        out_type=jax.ShapeDtypeStruct((rows, dim), dtype),
        scratch_types=(
            [pltpu.VMEM((_CHUNK_ROWS, dim), dtype)] * _NBUF
            + [pltpu.SemaphoreType.DMA] * (2 * _NBUF)
        ),
    )
    def sc_copy(tab, out, *refs):
        bufs = refs[:_NBUF]
        lsems = refs[_NBUF : 2 * _NBUF]
        ssems = refs[2 * _NBUF :]
        wid = lax.axis_index("s") * _NC + lax.axis_index("c")
        base = wid * rows_per_w

        def src(i):
            return tab.at[pl.ds(base + i * _CHUNK_ROWS, _CHUNK_ROWS)]

        def dst(i):
            return out.at[pl.ds(base + i * _CHUNK_ROWS, _CHUNK_ROWS)]

        loads = [None] * n_chunks
        stores = [None] * n_chunks
        for i in range(min(_NBUF, n_chunks)):
            loads[i] = pltpu.async_copy(src(i), bufs[i], lsems[i])
        for i in range(n_chunks):
            b = i % _NBUF
            loads[i].wait()
            stores[i] = pltpu.async_copy(bufs[b], dst(i), ssems[b])
            nxt = i + _NBUF
            if nxt < n_chunks:
                stores[i].wait()
                loads[nxt] = pltpu.async_copy(src(nxt), bufs[b], lsems[b])
        for i in range(max(0, n_chunks - _NBUF), n_chunks):
            if stores[i] is not None and i + _NBUF >= n_chunks:
                stores[i].wait()

    return sc_copy


def _tc_body(src, dst):
    dst[...] = src[...]


def _tc_copy_head(out_rows, head_rows, dim, dtype, tab):
    return pl.pallas_call(
        _tc_body,
        out_shape=jax.ShapeDtypeStruct((out_rows, dim), dtype),
        grid=(head_rows // _BLOCK_ROWS,),
        in_specs=[pl.BlockSpec((_BLOCK_ROWS, dim), lambda i: (i, 0))],
        out_specs=pl.BlockSpec((_BLOCK_ROWS, dim), lambda i: (i, 0)),
    )(tab)


def kernel(x, emb_weight):
    sl = x.shape[1]
    dim = emb_weight.shape[1]
    dtype = emb_weight.dtype
    tc_rows = sl - _SC_ROWS
    sc_part = _make_sc_copy(_SC_ROWS, dim, dtype)(emb_weight[tc_rows:sl])
    tc_full = _tc_copy_head(sl, tc_rows, dim, dtype, emb_weight[:tc_rows])
    return lax.dynamic_update_slice(tc_full, sc_part, (tc_rows, 0))


# speedup vs baseline: 1.5917x; 1.3498x over previous
import functools

import jax
import jax.numpy as jnp
from jax import lax
from jax.experimental import pallas as pl
from jax.experimental.pallas import tpu as pltpu
from jax.experimental.pallas import tpu_sc as plsc

_INFO = plsc.get_sparse_core_info()
_NC, _NS = _INFO.num_cores, _INFO.num_subcores
_NW = _NC * _NS
_CHUNK_ROWS = 16
_NBUF = 3
_SC_ROWS = 2048
_BLOCK_ROWS = 1024


def _make_sc_copy(rows, dim, dtype, out_rows=None, out_base=0):
    if out_rows is None:
        out_rows = rows
    rows_per_w = rows // _NW
    n_chunks = rows_per_w // _CHUNK_ROWS
    mesh = plsc.VectorSubcoreMesh(core_axis_name="c", subcore_axis_name="s")

    @functools.partial(
        pl.kernel,
        mesh=mesh,
        out_type=jax.ShapeDtypeStruct((out_rows, dim), dtype),
        scratch_types=(
            [pltpu.VMEM((_CHUNK_ROWS, dim), dtype)] * _NBUF
            + [pltpu.SemaphoreType.DMA] * (2 * _NBUF)
        ),
    )
    def sc_copy(tab, out, *refs):
        bufs = refs[:_NBUF]
        lsems = refs[_NBUF : 2 * _NBUF]
        ssems = refs[2 * _NBUF :]
        wid = lax.axis_index("s") * _NC + lax.axis_index("c")
        base = wid * rows_per_w

        def src(i):
            return tab.at[pl.ds(out_base + base + i * _CHUNK_ROWS, _CHUNK_ROWS)]

        def dst(i):
            return out.at[pl.ds(out_base + base + i * _CHUNK_ROWS, _CHUNK_ROWS)]

        loads = [None] * n_chunks
        stores = [None] * n_chunks
        for i in range(min(_NBUF, n_chunks)):
            loads[i] = pltpu.async_copy(src(i), bufs[i], lsems[i])
        for i in range(n_chunks):
            b = i % _NBUF
            loads[i].wait()
            stores[i] = pltpu.async_copy(bufs[b], dst(i), ssems[b])
            nxt = i + _NBUF
            if nxt < n_chunks:
                stores[i].wait()
                loads[nxt] = pltpu.async_copy(src(nxt), bufs[b], lsems[b])
        for i in range(max(0, n_chunks - _NBUF), n_chunks):
            if stores[i] is not None and i + _NBUF >= n_chunks:
                stores[i].wait()

    return sc_copy


def _tc_body(src, dst):
    dst[...] = src[...]


def _tc_body_alias(src, _carry, dst):
    dst[...] = src[...]


def _tc_copy_head_into(out_rows, head_rows, dim, dtype, tab, carry):
    return pl.pallas_call(
        _tc_body_alias,
        out_shape=jax.ShapeDtypeStruct((out_rows, dim), dtype),
        grid=(head_rows // _BLOCK_ROWS,),
        in_specs=[
            pl.BlockSpec((_BLOCK_ROWS, dim), lambda i: (i, 0)),
            pl.BlockSpec(memory_space=pl.ANY),
        ],
        out_specs=pl.BlockSpec((_BLOCK_ROWS, dim), lambda i: (i, 0)),
        input_output_aliases={1: 0},
    )(tab, carry)


def kernel(x, emb_weight):
    sl = x.shape[1]
    dim = emb_weight.shape[1]
    dtype = emb_weight.dtype
    tc_rows = sl - _SC_ROWS
    sc_full = _make_sc_copy(
        _SC_ROWS, dim, dtype, out_rows=sl, out_base=tc_rows
    )(emb_weight)
    return _tc_copy_head_into(sl, tc_rows, dim, dtype, emb_weight[:tc_rows], sc_full)


# speedup vs baseline: 2.2289x; 1.4003x over previous
import functools

import jax
import jax.numpy as jnp
from jax import lax
from jax.experimental import pallas as pl
from jax.experimental.pallas import tpu as pltpu
from jax.experimental.pallas import tpu_sc as plsc

_INFO = plsc.get_sparse_core_info()
_NC, _NS = _INFO.num_cores, _INFO.num_subcores
_NW = _NC * _NS
_CHUNK_ROWS = 16
_NBUF = 3


def _make_sc_copy(sl, dim, dtype):
    rows_per_w = sl // _NW
    n_chunks = rows_per_w // _CHUNK_ROWS
    mesh = plsc.VectorSubcoreMesh(core_axis_name="c", subcore_axis_name="s")

    @functools.partial(
        pl.kernel,
        mesh=mesh,
        out_type=jax.ShapeDtypeStruct((sl, dim), dtype),
        scratch_types=(
            [pltpu.VMEM_SHARED((_NS, _CHUNK_ROWS, dim), dtype)] * _NBUF
            + [pltpu.SemaphoreType.DMA] * (2 * _NBUF)
        ),
    )
    def sc_copy(tab, out, *refs):
        sid = lax.axis_index("s")
        bufs = [r.at[sid] for r in refs[:_NBUF]]
        lsems = refs[_NBUF : 2 * _NBUF]
        ssems = refs[2 * _NBUF :]
        wid = sid * _NC + lax.axis_index("c")
        base = wid * rows_per_w

        def src(i):
            return tab.at[pl.ds(base + i * _CHUNK_ROWS, _CHUNK_ROWS)]

        def dst(i):
            return out.at[pl.ds(base + i * _CHUNK_ROWS, _CHUNK_ROWS)]

        loads = [None] * n_chunks
        stores = [None] * n_chunks
        for i in range(min(_NBUF, n_chunks)):
            loads[i] = pltpu.async_copy(src(i), bufs[i], lsems[i])
        for i in range(n_chunks):
            b = i % _NBUF
            loads[i].wait()
            stores[i] = pltpu.async_copy(bufs[b], dst(i), ssems[b])
            nxt = i + _NBUF
            if nxt < n_chunks:
                stores[i].wait()
                loads[nxt] = pltpu.async_copy(src(nxt), bufs[b], lsems[b])
        for i in range(max(0, n_chunks - _NBUF), n_chunks):
            if stores[i] is not None and i + _NBUF >= n_chunks:
                stores[i].wait()

    return sc_copy


def kernel(x, emb_weight):
    sl = x.shape[1]
    dim = emb_weight.shape[1]
    return _make_sc_copy(sl, dim, emb_weight.dtype)(emb_weight[:sl])


# speedup vs baseline: 2.2305x; 1.0007x over previous
import functools

import jax
from jax import lax
from jax.experimental import pallas as pl
from jax.experimental.pallas import tpu as pltpu
from jax.experimental.pallas import tpu_sc as plsc

_INFO = plsc.get_sparse_core_info()
_NC, _NS = _INFO.num_cores, _INFO.num_subcores
_NW = _NC * _NS
_CHUNK_ROWS = 16
_NBUF = 3


def _make_sc_copy(sl, dim, dtype):
    rows_per_w = sl // _NW
    n_chunks = rows_per_w // _CHUNK_ROWS
    mesh = plsc.VectorSubcoreMesh(core_axis_name="c", subcore_axis_name="s")

    @functools.partial(
        pl.kernel,
        mesh=mesh,
        out_type=jax.ShapeDtypeStruct((sl, dim), dtype),
        scratch_types=(
            [pltpu.VMEM_SHARED((_NS, _CHUNK_ROWS, dim), dtype)] * _NBUF
            + [pltpu.SemaphoreType.DMA] * (2 * _NBUF)
        ),
    )
    def sc_copy(tab, out, *refs):
        sid = lax.axis_index("s")
        bufs = [r.at[sid] for r in refs[:_NBUF]]
        lsems = refs[_NBUF : 2 * _NBUF]
        ssems = refs[2 * _NBUF :]
        wid = sid * _NC + lax.axis_index("c")
        base = wid * rows_per_w

        def src(i):
            return tab.at[pl.ds(base + i * _CHUNK_ROWS, _CHUNK_ROWS)]

        def dst(i):
            return out.at[pl.ds(base + i * _CHUNK_ROWS, _CHUNK_ROWS)]

        loads = [None] * n_chunks
        stores = [None] * n_chunks
        for i in range(min(_NBUF, n_chunks)):
            loads[i] = pltpu.async_copy(src(i), bufs[i], lsems[i])
        for i in range(n_chunks):
            b = i % _NBUF
            loads[i].wait()
            stores[i] = pltpu.async_copy(bufs[b], dst(i), ssems[b])
            nxt = i + _NBUF
            if nxt < n_chunks:
                stores[i].wait()
                loads[nxt] = pltpu.async_copy(src(nxt), bufs[b], lsems[b])
        for i in range(max(0, n_chunks - _NBUF), n_chunks):
            if stores[i] is not None and i + _NBUF >= n_chunks:
                stores[i].wait()

    return sc_copy


def kernel(x, emb_weight):
    sl = x.shape[1]
    dim = emb_weight.shape[1]
    return _make_sc_copy(sl, dim, emb_weight.dtype)(emb_weight[:sl])
